# baseline (speedup 1.0000x reference)
import functools

import jax
import jax.numpy as jnp
from jax.experimental import pallas as pl
from jax.experimental.pallas import tpu as pltpu

_SEG_TILE = 256
_MAX_TILES = 2


def _pool_body(t0_ref, nt_ref, flat_ref, lo_ref, hi_ref, out_ref, t_ref,
               *, block_tok, num_sents):
    b = pl.program_id(0)

    @pl.when(b == 0)
    def _():
        out_ref[...] = jnp.zeros_like(out_ref)
        t_ref[...] = jax.lax.broadcasted_iota(
            jnp.int32, (_SEG_TILE, block_tok), 1)

    fb = flat_ref[...].astype(jnp.bfloat16)
    t = t_ref[...]
    off = b * block_tok
    t0 = t0_ref[b]
    nt = nt_ref[b]
    for j in range(_MAX_TILES):
        @pl.when(j < nt)
        def _():
            w = jnp.minimum(t0 + j, _MAX_TILES - 1)
            lo = lo_ref[w] - off
            hi = hi_ref[w] - off
            onehot = jnp.logical_and(t >= lo, t < hi).astype(jnp.bfloat16)
            part = jax.lax.dot_general(
                onehot, fb, (((1,), (0,)), ((), ())),
                preferred_element_type=jnp.float32)
            out_ref[w] += part


def _head_body(sums_ref, inv_ref, we_ref, wc_ref, out_ref):
    num_sents, ntags = out_ref.shape
    sums = sums_ref[...].reshape(num_sents, we_ref.shape[1])
    mean = sums * inv_ref[...]
    combined = jax.lax.dot_general(
        wc_ref[...], we_ref[...], (((1,), (0,)), ((), ())),
        precision=jax.lax.Precision.HIGHEST,
        preferred_element_type=jnp.float32)
    logits = jax.lax.dot_general(
        mean, combined, (((1,), (1,)), ((), ())),
        precision=jax.lax.Precision.HIGHEST,
        preferred_element_type=jnp.float32)
    m = jnp.max(logits, axis=-1, keepdims=True)
    sh = logits - m
    lse = jnp.log(jnp.sum(jnp.exp(sh), axis=-1, keepdims=True))
    out_ref[...] = sh - lse


def kernel(flat, cu_seqlens, W_e, W_c):
    total_tok, emb = flat.shape
    num_sents = cu_seqlens.shape[0] - 1
    ntags = W_c.shape[0]

    cu = cu_seqlens.astype(jnp.int32)
    cu_lo = cu[:-1].reshape(_MAX_TILES, _SEG_TILE, 1)
    cu_hi = cu[1:].reshape(_MAX_TILES, _SEG_TILE, 1)
    inv = 1.0 / jnp.maximum(cu[1:] - cu[:-1], 1).astype(jnp.float32)
    inv = inv.reshape(num_sents, 1)

    block_tok = 2048
    num_blocks = total_tok // block_tok

    starts = jnp.arange(num_blocks, dtype=jnp.int32) * block_tok
    first_seg = jnp.clip(
        jnp.searchsorted(cu, starts, side="right").astype(jnp.int32) - 1,
        0, num_sents - 1)
    last_seg = jnp.clip(
        jnp.searchsorted(cu, starts + (block_tok - 1), side="right")
        .astype(jnp.int32) - 1, 0, num_sents - 1)
    tile0 = first_seg // _SEG_TILE
    ntiles = last_seg // _SEG_TILE - tile0 + 1

    pool = functools.partial(_pool_body, block_tok=block_tok,
                             num_sents=num_sents)
    sums = pl.pallas_call(
        pool,
        grid_spec=pltpu.PrefetchScalarGridSpec(
            num_scalar_prefetch=2,
            grid=(num_blocks,),
            in_specs=[
                pl.BlockSpec((block_tok, emb), lambda b, *_: (b, 0)),
                pl.BlockSpec((_MAX_TILES, _SEG_TILE, 1),
                             lambda b, *_: (0, 0, 0)),
                pl.BlockSpec((_MAX_TILES, _SEG_TILE, 1),
                             lambda b, *_: (0, 0, 0)),
            ],
            out_specs=pl.BlockSpec((_MAX_TILES, _SEG_TILE, emb),
                                   lambda b, *_: (0, 0, 0)),
            scratch_shapes=[pltpu.VMEM((_SEG_TILE, block_tok), jnp.int32)],
        ),
        out_shape=jax.ShapeDtypeStruct((_MAX_TILES, _SEG_TILE, emb),
                                       jnp.float32),
    )(tile0, ntiles, flat, cu_lo, cu_hi)

    out = pl.pallas_call(
        _head_body,
        in_specs=[
            pl.BlockSpec((_MAX_TILES, _SEG_TILE, emb), lambda: (0, 0, 0)),
            pl.BlockSpec((num_sents, 1), lambda: (0, 0)),
            pl.BlockSpec((emb, emb), lambda: (0, 0)),
            pl.BlockSpec((ntags, emb), lambda: (0, 0)),
        ],
        out_specs=pl.BlockSpec((num_sents, ntags), lambda: (0, 0)),
        out_shape=jax.ShapeDtypeStruct((num_sents, ntags), jnp.float32),
    )(sums, inv, W_e, W_c)
    return out


# speedup vs baseline: 1.4735x; 1.4735x over previous
import functools

import jax
import jax.numpy as jnp
from jax.experimental import pallas as pl
from jax.experimental.pallas import tpu as pltpu


def _body(flat_ref, lo_ref, hi_ref, inv_ref, we_ref, wc_ref, out_ref, acc_ref,
          *, block_tok, num_blocks, num_sents):
    b = pl.program_id(0)
    t = jax.lax.broadcasted_iota(jnp.int32, (num_sents, block_tok), 1) + b * block_tok
    onehot = jnp.logical_and(t >= lo_ref[...], t < hi_ref[...]).astype(jnp.bfloat16)
    part = jax.lax.dot_general(
        onehot, flat_ref[...].astype(jnp.bfloat16),
        (((1,), (0,)), ((), ())), preferred_element_type=jnp.float32)

    @pl.when(b == 0)
    def _():
        acc_ref[...] = part

    @pl.when(b > 0)
    def _():
        acc_ref[...] += part

    @pl.when(b == num_blocks - 1)
    def _():
        mean = acc_ref[...] * inv_ref[...]
        combined = jax.lax.dot_general(
            wc_ref[...], we_ref[...], (((1,), (0,)), ((), ())),
            precision=jax.lax.Precision.HIGHEST,
            preferred_element_type=jnp.float32)
        logits = jax.lax.dot_general(
            mean, combined, (((1,), (1,)), ((), ())),
            precision=jax.lax.Precision.HIGHEST,
            preferred_element_type=jnp.float32)
        m = jnp.max(logits, axis=-1, keepdims=True)
        sh = logits - m
        lse = jnp.log(jnp.sum(jnp.exp(sh), axis=-1, keepdims=True))
        out_ref[...] = sh - lse


def kernel(flat, cu_seqlens, W_e, W_c):
    total_tok, emb = flat.shape
    num_sents = cu_seqlens.shape[0] - 1
    ntags = W_c.shape[0]
    cu = cu_seqlens.astype(jnp.int32)
    cu_lo = cu[:-1].reshape(num_sents, 1)
    cu_hi = cu[1:].reshape(num_sents, 1)
    inv = 1.0 / jnp.maximum(cu_hi - cu_lo, 1).astype(jnp.float32)

    block_tok = 4096
    num_blocks = total_tok // block_tok

    body = functools.partial(_body, block_tok=block_tok,
                             num_blocks=num_blocks, num_sents=num_sents)

    out = pl.pallas_call(
        body,
        grid=(num_blocks,),
        in_specs=[
            pl.BlockSpec((block_tok, emb), lambda b: (b, 0)),
            pl.BlockSpec((num_sents, 1), lambda b: (0, 0)),
            pl.BlockSpec((num_sents, 1), lambda b: (0, 0)),
            pl.BlockSpec((num_sents, 1), lambda b: (0, 0)),
            pl.BlockSpec((emb, emb), lambda b: (0, 0)),
            pl.BlockSpec((ntags, emb), lambda b: (0, 0)),
        ],
        out_specs=pl.BlockSpec((num_sents, ntags), lambda b: (0, 0)),
        out_shape=jax.ShapeDtypeStruct((num_sents, ntags), jnp.float32),
        scratch_shapes=[pltpu.VMEM((num_sents, emb), jnp.float32)],
    )(flat, cu_lo, cu_hi, inv, W_e, W_c)
    return out
